# baseline (speedup 1.0000x reference)
import dataclasses
import functools

import jax
import jax.numpy as jnp
from jax import lax
from jax.experimental import pallas as pl
from jax.experimental.pallas import tpu as pltpu
from jax.experimental.pallas import tpu_sc as plsc

B, N1, N2, NS = 4, 4096, 4096, 3
QT = 256
LANES = 16
NW = 32
NQ_W = B * N2 // NW
W_PER_B = N2 // NQ_W
TBL_W = N1 * LANES


def _knn_body(p1_ref, f1_ref, p2_ref, i_ref, w_ref):
    keys = p1_ref[0] + f1_ref[0]
    q = p2_ref[0]
    dx = q[:, 0:1] - keys[0:1, :]
    dy = q[:, 1:2] - keys[1:2, :]
    dz = q[:, 2:3] - keys[2:3, :]
    d2ex = dx * dx + dy * dy + dz * dz
    q2 = jnp.sum(q * q, axis=1, keepdims=True)
    k2 = (keys[0:1, :] * keys[0:1, :]
          + keys[1:2, :] * keys[1:2, :]
          + keys[2:3, :] * keys[2:3, :])
    dotn = lax.dot_general(q, -2.0 * keys, (((1,), (0,)), ((), ())),
                           preferred_element_type=jnp.float32)
    d2 = (q2 + k2) + dotn
    iotaf = lax.broadcasted_iota(jnp.int32, (QT, N1), 1).astype(jnp.float32)
    big = jnp.float32(3.0e38)

    def pick(d, last=False):
        m = jnp.min(d, axis=1, keepdims=True)
        eq = d == m
        fi = jnp.min(jnp.where(eq, iotaf, big), axis=1, keepdims=True)
        mex = jnp.min(jnp.where(eq, d2ex, big), axis=1, keepdims=True)
        i = fi.astype(jnp.int32)
        dn = None if last else jnp.where(eq, big, d)
        return i, mex, dn

    i1, m1, d2 = pick(d2)
    i2, m2, d2 = pick(d2)
    i3, m3, _ = pick(d2, last=True)

    r1 = 1.0 / jnp.maximum(jnp.sqrt(m1), 1e-10)
    r2 = 1.0 / jnp.maximum(jnp.sqrt(m2), 1e-10)
    r3 = 1.0 / jnp.maximum(jnp.sqrt(m3), 1e-10)
    norm = r1 + r2 + r3

    icat = jnp.concatenate([i1, i2, i3], axis=1) * LANES
    wcat = jnp.concatenate([r1 / norm, r2 / norm, r3 / norm], axis=1)
    i_ref[0] = jnp.swapaxes(icat, 0, 1)
    w_ref[0] = jnp.swapaxes(wcat, 0, 1)


def _knn(pos1, flow1, p2t):
    grid = (B, N2 // QT)
    return pl.pallas_call(
        _knn_body,
        grid=grid,
        in_specs=[
            pl.BlockSpec((1, 3, N1), lambda b, qi: (b, 0, 0)),
            pl.BlockSpec((1, 3, N1), lambda b, qi: (b, 0, 0)),
            pl.BlockSpec((1, QT, 3), lambda b, qi: (b, qi, 0)),
        ],
        out_specs=[pl.BlockSpec((1, 3, QT), lambda b, qi: (b, 0, qi))] * 2,
        out_shape=[jax.ShapeDtypeStruct((B, NS, N2), jnp.int32),
                   jax.ShapeDtypeStruct((B, NS, N2), jnp.float32)],
        compiler_params=pltpu.CompilerParams(
            dimension_semantics=("parallel", "arbitrary")),
    )(pos1, flow1, p2t)


def _sc_warp(tblf, idxf, wf, p2f):
    mesh = plsc.VectorSubcoreMesh(core_axis_name="c", subcore_axis_name="s")
    cp = pltpu.CompilerParams()
    if "needs_layout_passes" in pltpu.CompilerParams.__dataclass_fields__:
        cp = dataclasses.replace(cp, needs_layout_passes=False)

    @functools.partial(
        pl.kernel,
        mesh=mesh,
        compiler_params=cp,
        out_type=jax.ShapeDtypeStruct((B * NS * N2,), jnp.float32),
        scratch_types=[
            pltpu.VMEM((TBL_W,), jnp.float32),
            pltpu.VMEM((NQ_W,), jnp.int32),
            pltpu.VMEM((NQ_W,), jnp.int32),
            pltpu.VMEM((NQ_W,), jnp.int32),
            pltpu.VMEM((NQ_W,), jnp.float32),
            pltpu.VMEM((NQ_W,), jnp.float32),
            pltpu.VMEM((NQ_W,), jnp.float32),
            pltpu.VMEM((NS * NQ_W,), jnp.float32),
            pltpu.VMEM((NS * NQ_W,), jnp.float32),
            pltpu.SemaphoreType.DMA,
        ],
    )
    def k(tbl_hbm, i_hbm, w_hbm, p2_hbm,
          out_hbm, tbl_v, i1_v, i2_v, i3_v, w1_v, w2_v, w3_v, p2_v, out_v,
          sem):
        wid = lax.axis_index("s") * 2 + lax.axis_index("c")
        qbase = wid * NQ_W
        b = wid // W_PER_B
        qoff = qbase - b * N2

        copies = [
            pltpu.async_copy(tbl_hbm.at[pl.ds(b * TBL_W, TBL_W)], tbl_v, sem),
        ]
        for d, (iv, wv) in enumerate(((i1_v, w1_v), (i2_v, w2_v),
                                      (i3_v, w3_v))):
            row = (b * NS + d) * N2 + qoff
            copies.append(pltpu.async_copy(i_hbm.at[pl.ds(row, NQ_W)], iv, sem))
            copies.append(pltpu.async_copy(w_hbm.at[pl.ds(row, NQ_W)], wv, sem))
            copies.append(pltpu.async_copy(
                p2_hbm.at[pl.ds(row, NQ_W)],
                p2_v.at[pl.ds(d * NQ_W, NQ_W)], sem))
        for c in copies:
            c.wait()

        @pl.loop(0, NQ_W, step=LANES)
        def _(j):
            s = pl.ds(j, LANES)
            a1 = i1_v[s]
            a2 = i2_v[s]
            a3 = i3_v[s]
            w1 = w1_v[s]
            w2 = w2_v[s]
            w3 = w3_v[s]
            for d in range(NS):
                g1 = plsc.load_gather(tbl_v, [a1 + d])
                g2 = plsc.load_gather(tbl_v, [a2 + d])
                g3 = plsc.load_gather(tbl_v, [a3 + d])
                f = w1 * g1 + w2 * g2 + w3 * g3
                res = p2_v[pl.ds(d * NQ_W + j, LANES)] - f
                res = jnp.minimum(jnp.maximum(res, -10.0), 10.0)
                out_v[pl.ds(d * NQ_W + j, LANES)] = res

        out_copies = []
        for d in range(NS):
            out_copies.append(pltpu.async_copy(
                out_v.at[pl.ds(d * NQ_W, NQ_W)],
                out_hbm.at[pl.ds((b * NS + d) * N2 + qoff, NQ_W)], sem))
        for c in out_copies:
            c.wait()

    return k(tblf, idxf, wf, p2f)


def kernel(pos1, pos2, flow1):
    p2t = jnp.transpose(pos2, (0, 2, 1))
    iarr, warr = _knn(pos1, flow1, p2t)
    f1t = jnp.transpose(flow1, (0, 2, 1)).reshape(B * N1, NS)
    tblf = jnp.pad(f1t, ((0, 0), (0, LANES - NS))).reshape(-1)
    out = _sc_warp(tblf, iarr.reshape(-1), warr.reshape(-1), pos2.reshape(-1))
    return out.reshape(B, NS, N2)


# speedup vs baseline: 1.5419x; 1.5419x over previous
import dataclasses
import functools

import jax
import jax.numpy as jnp
from jax import lax
from jax.experimental import pallas as pl
from jax.experimental.pallas import tpu as pltpu
from jax.experimental.pallas import tpu_sc as plsc

B, N1, N2, NS = 4, 4096, 4096, 3
QT = 256
LANES = 16
NW = 32
NQ_W = B * N2 // NW
W_PER_B = N2 // NQ_W
TBL_W = N1 * LANES


def _knn_body(p1_ref, f1_ref, p2_ref, i_ref, w_ref):
    keys = p1_ref[0] + f1_ref[0]
    q = p2_ref[0]
    q2 = jnp.sum(q * q, axis=1, keepdims=True)
    k2 = (keys[0:1, :] * keys[0:1, :]
          + keys[1:2, :] * keys[1:2, :]
          + keys[2:3, :] * keys[2:3, :])
    dotn = lax.dot_general(q, -2.0 * keys, (((1,), (0,)), ((), ())),
                           preferred_element_type=jnp.float32)
    d2 = (q2 + k2) + dotn
    iotaf = lax.broadcasted_iota(jnp.int32, (QT, N1), 1).astype(jnp.float32)
    big = jnp.float32(3.0e38)

    def pick(d, last=False):
        m = jnp.min(d, axis=1, keepdims=True)
        eq = d == m
        fi = jnp.min(jnp.where(eq, iotaf, big), axis=1, keepdims=True)
        i = fi.astype(jnp.int32)
        dn = None if last else jnp.where(eq, big, d)
        return i, m, dn

    i1, m1, d2 = pick(d2)
    i2, m2, d2 = pick(d2)
    i3, m3, _ = pick(d2, last=True)

    r1 = 1.0 / jnp.maximum(jnp.sqrt(jnp.maximum(m1, 0.0)), 1e-10)
    r2 = 1.0 / jnp.maximum(jnp.sqrt(jnp.maximum(m2, 0.0)), 1e-10)
    r3 = 1.0 / jnp.maximum(jnp.sqrt(jnp.maximum(m3, 0.0)), 1e-10)
    norm = r1 + r2 + r3

    icat = jnp.concatenate([i1, i2, i3], axis=1) * LANES
    wcat = jnp.concatenate([r1 / norm, r2 / norm, r3 / norm], axis=1)
    i_ref[0] = jnp.swapaxes(icat, 0, 1)
    w_ref[0] = jnp.swapaxes(wcat, 0, 1)


def _knn(pos1, flow1, p2t):
    grid = (B, N2 // QT)
    return pl.pallas_call(
        _knn_body,
        grid=grid,
        in_specs=[
            pl.BlockSpec((1, 3, N1), lambda b, qi: (b, 0, 0)),
            pl.BlockSpec((1, 3, N1), lambda b, qi: (b, 0, 0)),
            pl.BlockSpec((1, QT, 3), lambda b, qi: (b, qi, 0)),
        ],
        out_specs=[pl.BlockSpec((1, 3, QT), lambda b, qi: (b, 0, qi))] * 2,
        out_shape=[jax.ShapeDtypeStruct((B, NS, N2), jnp.int32),
                   jax.ShapeDtypeStruct((B, NS, N2), jnp.float32)],
        compiler_params=pltpu.CompilerParams(
            dimension_semantics=("parallel", "arbitrary")),
    )(pos1, flow1, p2t)


def _sc_warp(tblf, idxf, wf, p2f):
    mesh = plsc.VectorSubcoreMesh(core_axis_name="c", subcore_axis_name="s")
    cp = pltpu.CompilerParams()
    if "needs_layout_passes" in pltpu.CompilerParams.__dataclass_fields__:
        cp = dataclasses.replace(cp, needs_layout_passes=False)

    @functools.partial(
        pl.kernel,
        mesh=mesh,
        compiler_params=cp,
        out_type=jax.ShapeDtypeStruct((B * NS * N2,), jnp.float32),
        scratch_types=[
            pltpu.VMEM((TBL_W,), jnp.float32),
            pltpu.VMEM((NQ_W,), jnp.int32),
            pltpu.VMEM((NQ_W,), jnp.int32),
            pltpu.VMEM((NQ_W,), jnp.int32),
            pltpu.VMEM((NQ_W,), jnp.float32),
            pltpu.VMEM((NQ_W,), jnp.float32),
            pltpu.VMEM((NQ_W,), jnp.float32),
            pltpu.VMEM((NS * NQ_W,), jnp.float32),
            pltpu.VMEM((NS * NQ_W,), jnp.float32),
            pltpu.SemaphoreType.DMA,
        ],
    )
    def k(tbl_hbm, i_hbm, w_hbm, p2_hbm,
          out_hbm, tbl_v, i1_v, i2_v, i3_v, w1_v, w2_v, w3_v, p2_v, out_v,
          sem):
        wid = lax.axis_index("s") * 2 + lax.axis_index("c")
        qbase = wid * NQ_W
        b = wid // W_PER_B
        qoff = qbase - b * N2

        copies = [
            pltpu.async_copy(tbl_hbm.at[pl.ds(b * TBL_W, TBL_W)], tbl_v, sem),
        ]
        for d, (iv, wv) in enumerate(((i1_v, w1_v), (i2_v, w2_v),
                                      (i3_v, w3_v))):
            row = (b * NS + d) * N2 + qoff
            copies.append(pltpu.async_copy(i_hbm.at[pl.ds(row, NQ_W)], iv, sem))
            copies.append(pltpu.async_copy(w_hbm.at[pl.ds(row, NQ_W)], wv, sem))
            copies.append(pltpu.async_copy(
                p2_hbm.at[pl.ds(row, NQ_W)],
                p2_v.at[pl.ds(d * NQ_W, NQ_W)], sem))
        for c in copies:
            c.wait()

        @pl.loop(0, NQ_W, step=LANES)
        def _(j):
            s = pl.ds(j, LANES)
            a1 = i1_v[s]
            a2 = i2_v[s]
            a3 = i3_v[s]
            w1 = w1_v[s]
            w2 = w2_v[s]
            w3 = w3_v[s]
            for d in range(NS):
                g1 = plsc.load_gather(tbl_v, [a1 + d])
                g2 = plsc.load_gather(tbl_v, [a2 + d])
                g3 = plsc.load_gather(tbl_v, [a3 + d])
                f = w1 * g1 + w2 * g2 + w3 * g3
                res = p2_v[pl.ds(d * NQ_W + j, LANES)] - f
                res = jnp.minimum(jnp.maximum(res, -10.0), 10.0)
                out_v[pl.ds(d * NQ_W + j, LANES)] = res

        out_copies = []
        for d in range(NS):
            out_copies.append(pltpu.async_copy(
                out_v.at[pl.ds(d * NQ_W, NQ_W)],
                out_hbm.at[pl.ds((b * NS + d) * N2 + qoff, NQ_W)], sem))
        for c in out_copies:
            c.wait()

    return k(tblf, idxf, wf, p2f)


def kernel(pos1, pos2, flow1):
    p2t = jnp.transpose(pos2, (0, 2, 1))
    iarr, warr = _knn(pos1, flow1, p2t)
    f1t = jnp.transpose(flow1, (0, 2, 1)).reshape(B * N1, NS)
    tblf = jnp.pad(f1t, ((0, 0), (0, LANES - NS))).reshape(-1)
    out = _sc_warp(tblf, iarr.reshape(-1), warr.reshape(-1), pos2.reshape(-1))
    return out.reshape(B, NS, N2)
